# baseline (speedup 1.0000x reference)
import functools

import numpy as np
import jax
import jax.numpy as jnp
from jax import lax
from jax.experimental import pallas as pl
from jax.experimental.pallas import tpu as pltpu
from jax.experimental.pallas import tpu_sc as plsc

_D = 256
_NH = 8
_NL = 4
_NP = 4
_HD = _D // _NH
_NPTS = _NL * _NP
_NK = _NPTS * 4
_SPATIAL = [(64, 64), (32, 32), (16, 16), (8, 8)]
_STARTS = [0, 4096, 5120, 5376]
_LEN = 5440
_B = 2
_LQ = 5440

_RB = 544
_NRB = _LQ // _RB

_CHQ = 80
_FL = 4
_QHALF = _LQ // 2
_NCH = _QHALF // _CHQ


def _t1_body(x_ref, w_ref, b_ref, out_ref):
    v = jnp.dot(x_ref[0].astype(jnp.bfloat16), w_ref[...].astype(jnp.bfloat16),
                preferred_element_type=jnp.float32)
    v = v + b_ref[...]
    lo = v[:, :128]
    hi = v[:, 128:]
    bl = lax.bitcast_convert_type(lo, jnp.int32)
    bh = lax.bitcast_convert_type(hi, jnp.int32)
    mask = jnp.int32(-65536)
    sh16 = jnp.full(bl.shape, 16, jnp.int32)
    rl = (bl + 32768) & mask
    rh = (bh + 32768) & mask
    word = rh | lax.shift_right_logical(rl, sh16)
    for h in range(_NH):
        out_ref[0, h, :, :] = word[:, h * 16:(h + 1) * 16]


def _t1(input_flatten, w_perm_t, b_perm):
    return pl.pallas_call(
        _t1_body,
        grid=(_B, _LEN // _RB),
        in_specs=[
            pl.BlockSpec((1, _RB, _D), lambda b, r: (b, r, 0)),
            pl.BlockSpec((_D, _D), lambda b, r: (0, 0)),
            pl.BlockSpec((1, _D), lambda b, r: (0, 0)),
        ],
        out_specs=pl.BlockSpec((1, _NH, _RB, 16), lambda b, r: (b, 0, r, 0)),
        out_shape=jax.ShapeDtypeStruct((_B, _NH, _LEN, 16), jnp.int32),
    )(input_flatten, w_perm_t, b_perm)


def _t2_body(q_ref, rp_ref, wc_ref, bc_ref, g_ref,
             scale_ref, maxc_ref, sl_ref, wl_ref, idx_ref, wt_ref):
    m = jnp.dot(q_ref[0].astype(jnp.bfloat16), wc_ref[...].astype(jnp.bfloat16),
                preferred_element_type=jnp.float32)
    m = m + bc_ref[...]
    xy = m[:, :256] + rp_ref[0]
    logits = m[:, 256:]

    loc = jnp.clip(xy, 0.0, 1.0) * scale_ref[...] - 0.5
    t0 = jnp.floor(loc)
    c0 = jnp.clip(t0, 0.0, maxc_ref[...])
    c1 = jnp.clip(t0 + 1.0, 0.0, maxc_ref[...])
    wa = c1 - loc
    wb = loc - c0

    x0 = c0[:, :128]
    y0 = c0[:, 128:]
    x1 = c1[:, :128]
    y1 = c1[:, 128:]
    ax = wa[:, :128]
    ay = wa[:, 128:]
    bx = wb[:, :128]
    by = wb[:, 128:]

    mx = jnp.max(logits, axis=-1, keepdims=True)
    e = jnp.exp(logits - mx)
    gs = jnp.dot(e, g_ref[...], preferred_element_type=jnp.float32)
    aw = e / gs

    sl = sl_ref[...]
    wl = wl_ref[...]
    ra = (sl + y0 * wl + x0) * 16.0
    rb = (sl + y1 * wl + x0) * 16.0
    rc = (sl + y0 * wl + x1) * 16.0
    rd = (sl + y1 * wl + x1) * 16.0
    pa = ax * ay * aw
    pb = ax * by * aw
    pc = bx * ay * aw
    pd = bx * by * aw

    corners = [(ra, pa), (rb, pb), (rc, pc), (rd, pd)]
    for h in range(_NH):
        s = slice(h * 16, (h + 1) * 16)
        for ci, (r, p) in enumerate(corners):
            idx_ref[0, h, :, ci * 16:(ci + 1) * 16] = r[:, s].astype(jnp.int32)
            wt_ref[0, h, :, ci * 16:(ci + 1) * 16] = p[:, s]


def _t2(query, rp_bcast, wc, bc, g, scale, maxc, sl, wl):
    return pl.pallas_call(
        _t2_body,
        grid=(_B, _NRB),
        in_specs=[
            pl.BlockSpec((1, _RB, _D), lambda b, r: (b, r, 0)),
            pl.BlockSpec((1, _RB, 256), lambda b, r: (b, r, 0)),
            pl.BlockSpec((_D, 384), lambda b, r: (0, 0)),
            pl.BlockSpec((1, 384), lambda b, r: (0, 0)),
            pl.BlockSpec((128, 128), lambda b, r: (0, 0)),
            pl.BlockSpec((1, 256), lambda b, r: (0, 0)),
            pl.BlockSpec((1, 256), lambda b, r: (0, 0)),
            pl.BlockSpec((1, 128), lambda b, r: (0, 0)),
            pl.BlockSpec((1, 128), lambda b, r: (0, 0)),
        ],
        out_specs=[
            pl.BlockSpec((1, _NH, _RB, 65), lambda b, r: (b, 0, r, 0)),
            pl.BlockSpec((1, _NH, _RB, 65), lambda b, r: (b, 0, r, 0)),
        ],
        out_shape=[
            jax.ShapeDtypeStruct((_B, _NH, _LQ, 65), jnp.int32),
            jax.ShapeDtypeStruct((_B, _NH, _LQ, 65), jnp.float32),
        ],
    )(query, rp_bcast, wc, bc, g, scale, maxc, sl, wl)


def _sc_body(tbl, idx, wt, out, tblv, idxv, wtv, outv):
    c = lax.axis_index("core")
    s = lax.axis_index("sub")
    b = c
    h = s // 2
    half = s % 2

    pltpu.sync_copy(tbl.at[b, h], tblv)

    iota16 = lax.iota(jnp.int32, 16)
    sh16 = jnp.full((16,), 16, jnp.int32)
    mask_hi = jnp.full((16,), -65536, jnp.int32)
    half_ulp = jnp.full((16,), 32768, jnp.int32)
    zf32 = jnp.zeros((16,), jnp.float32)
    zi32 = jnp.zeros((16,), jnp.int32)
    rots = [(iota16 + j) & 15 for j in range(16)]

    def chunk_body(ch, carry):
        qs = half * _QHALF + ch * _CHQ
        pltpu.sync_copy(idx.at[b, h, pl.ds(qs, _CHQ)], idxv)
        pltpu.sync_copy(wt.at[b, h, pl.ds(qs, _CHQ)], wtv)

        def qb_body(qb, carry2):
            qv = iota16 + qb * 16
            for d in range(_HD):
                plsc.store_scatter(outv, [qv, zi32 + d], zf32)

            def kg_body(kg, carry3):
                kbase = kg * _FL
                ks = [zi32 + (kbase + j) for j in range(_FL)]
                idxrs = [plsc.load_gather(idxv, [qv, k]) for k in ks]
                wtrs = [plsc.load_gather(wtv, [qv, k]) for k in ks]
                wtbs = []
                for w in wtrs:
                    wbits = plsc.bitcast(w, jnp.int32)
                    r = (wbits + half_ulp) & mask_hi
                    word = r | lax.shift_right_logical(r, sh16)
                    wtbs.append(plsc.bitcast(word, jnp.bfloat16))
                acc = [None] * 16
                for j in range(_FL):
                    wvs = [plsc.load_gather(tblv, [idxrs[j] + rots[s]])
                           for s in range(16)]
                    for s in range(16):
                        t = wtbs[j] * plsc.bitcast(wvs[s], jnp.bfloat16)
                        acc[s] = t if j == 0 else acc[s] + t
                for s in range(16):
                    u = plsc.bitcast(acc[s], jnp.int32)
                    fhi = plsc.bitcast(u & mask_hi, jnp.float32)
                    flo = plsc.bitcast(lax.shift_left(u, sh16), jnp.float32)
                    col = rots[s] + rots[s]
                    plsc.addupdate_scatter(outv, [qv, col], flo)
                    plsc.addupdate_scatter(outv, [qv, col + 1], fhi)
                return carry3

            lax.fori_loop(0, _NK // _FL, kg_body, 0)
            return carry2

        lax.fori_loop(0, _CHQ // 16, qb_body, 0)
        pltpu.sync_copy(outv, out.at[b, h, pl.ds(qs, _CHQ)])
        return carry

    lax.fori_loop(0, _NCH, chunk_body, 0)


@functools.cache
def _sc_sample():
    return pl.kernel(
        _sc_body,
        out_type=jax.ShapeDtypeStruct((_B, _NH, _LQ, 33), jnp.float32),
        mesh=plsc.VectorSubcoreMesh(core_axis_name="core",
                                    subcore_axis_name="sub",
                                    num_cores=2, num_subcores=16),
        compiler_params=pltpu.CompilerParams(needs_layout_passes=False),
        scratch_types=[
            pltpu.VMEM((_LEN * 16,), jnp.int32),
            pltpu.VMEM((_CHQ, 65), jnp.int32),
            pltpu.VMEM((_CHQ, 65), jnp.float32),
            pltpu.VMEM((_CHQ, 33), jnp.float32),
        ],
    )


def _t3_body(s_ref, w_ref, b_ref, out_ref):
    h = pl.program_id(2)
    acc = jnp.dot(s_ref[0, 0, :, :_HD].astype(jnp.bfloat16),
                  w_ref[...].astype(jnp.bfloat16),
                  preferred_element_type=jnp.float32)

    @pl.when(h == 0)
    def _():
        out_ref[0] = acc + b_ref[...]

    @pl.when(h != 0)
    def _():
        out_ref[0] = out_ref[0] + acc


def _t3(sampled, wo_t, b_o):
    return pl.pallas_call(
        _t3_body,
        grid=(_B, _NRB, _NH),
        in_specs=[
            pl.BlockSpec((1, 1, _RB, 33), lambda b, r, h: (b, h, r, 0)),
            pl.BlockSpec((_HD, _D), lambda b, r, h: (h, 0)),
            pl.BlockSpec((1, _D), lambda b, r, h: (0, 0)),
        ],
        out_specs=pl.BlockSpec((1, _RB, _D), lambda b, r, h: (b, r, 0)),
        out_shape=jax.ShapeDtypeStruct((_B, _LQ, _D), jnp.float32),
    )(sampled, wo_t, b_o.reshape(1, _D))


def _consts():
    perm = np.empty((256,), np.int64)
    for half in range(2):
        for h in range(_NH):
            for dp in range(16):
                perm[half * 128 + h * 16 + dp] = h * 32 + 2 * dp + half

    lanes = np.arange(128)
    lvl = (lanes % 16) // 4
    w_l = np.array([_SPATIAL[l][1] for l in lvl], np.float32)
    h_l = np.array([_SPATIAL[l][0] for l in lvl], np.float32)
    s_l = np.array([_STARTS[l] for l in lvl], np.float32)

    scale = np.concatenate([w_l, h_l]).reshape(1, 256)
    maxc = np.concatenate([w_l - 1, h_l - 1]).reshape(1, 256)
    sl = s_l.reshape(1, 128)
    wl = w_l.reshape(1, 128)

    scm = np.zeros((8, 384), np.float32)
    for lane in range(128):
        l = int(lvl[lane])
        scm[2 * l, lane] = 1.0
        scm[2 * l + 1, 128 + lane] = 1.0

    g = (np.arange(128)[:, None] // 16 == np.arange(128)[None, :] // 16)
    g = g.astype(np.float32)
    return perm, scale, maxc, sl, wl, scm, g


_PERM, _SCALE, _MAXC, _SL, _WL, _SCM, _G = _consts()


def kernel(query, reference_points, input_flatten, input_spatial_shapes,
           input_level_start_index, W_so, b_so, W_aw, b_aw, W_v, b_v,
           W_o, b_o):
    w_perm_t = W_v[_PERM, :].T
    b_perm = b_v[_PERM].reshape(1, _D)
    wc = jnp.concatenate([W_so[0::2], W_so[1::2], W_aw], axis=0).T
    bc = jnp.concatenate([b_so[0::2], b_so[1::2], b_aw]).reshape(1, 384)
    wo_t = W_o.T
    rp4 = jnp.repeat(reference_points, _NP, axis=2)
    rp_bcast = jnp.concatenate(
        [jnp.tile(rp4[..., 0], (1, 1, _NH)),
         jnp.tile(rp4[..., 1], (1, 1, _NH))], axis=-1)

    tblp = _t1(input_flatten, w_perm_t, b_perm)
    idx_a, wt_a = _t2(query, rp_bcast, wc, bc,
                      jnp.asarray(_G), jnp.asarray(_SCALE),
                      jnp.asarray(_MAXC), jnp.asarray(_SL), jnp.asarray(_WL))
    sampled = _sc_sample()(tblp.reshape(_B, _NH, _LEN * 16), idx_a, wt_a)
    return _t3(sampled, wo_t, b_o)


# speedup vs baseline: 1.0889x; 1.0889x over previous
import functools

import numpy as np
import jax
import jax.numpy as jnp
from jax import lax
from jax.experimental import pallas as pl
from jax.experimental.pallas import tpu as pltpu
from jax.experimental.pallas import tpu_sc as plsc

_D = 256
_NH = 8
_NL = 4
_NP = 4
_HD = _D // _NH
_NPTS = _NL * _NP
_NK = _NPTS * 4
_SPATIAL = [(64, 64), (32, 32), (16, 16), (8, 8)]
_STARTS = [0, 4096, 5120, 5376]
_LEN = 5440
_B = 2
_LQ = 5440

_RB = 544
_NRB = _LQ // _RB

_CHQ = 160
_FL = 4
_QHALF = _LQ // 2
_NCH = _QHALF // _CHQ


def _t1_body(x_ref, w_ref, b_ref, out_ref):
    v = jnp.dot(x_ref[0].astype(jnp.bfloat16), w_ref[...].astype(jnp.bfloat16),
                preferred_element_type=jnp.float32)
    v = v + b_ref[...]
    lo = v[:, :128]
    hi = v[:, 128:]
    bl = lax.bitcast_convert_type(lo, jnp.int32)
    bh = lax.bitcast_convert_type(hi, jnp.int32)
    mask = jnp.int32(-65536)
    sh16 = jnp.full(bl.shape, 16, jnp.int32)
    rl = (bl + 32768) & mask
    rh = (bh + 32768) & mask
    word = rh | lax.shift_right_logical(rl, sh16)
    for h in range(_NH):
        out_ref[0, h, :, :] = word[:, h * 16:(h + 1) * 16]


def _t1(input_flatten, w_perm_t, b_perm):
    return pl.pallas_call(
        _t1_body,
        grid=(_B, _LEN // _RB),
        in_specs=[
            pl.BlockSpec((1, _RB, _D), lambda b, r: (b, r, 0)),
            pl.BlockSpec((_D, _D), lambda b, r: (0, 0)),
            pl.BlockSpec((1, _D), lambda b, r: (0, 0)),
        ],
        out_specs=pl.BlockSpec((1, _NH, _RB, 16), lambda b, r: (b, 0, r, 0)),
        out_shape=jax.ShapeDtypeStruct((_B, _NH, _LEN, 16), jnp.int32),
    )(input_flatten, w_perm_t, b_perm)


def _t2_body(q_ref, rp_ref, wc_ref, bc_ref, g_ref,
             scale_ref, maxc_ref, sl_ref, wl_ref, idx_ref, wt_ref):
    m = jnp.dot(q_ref[0].astype(jnp.bfloat16), wc_ref[...].astype(jnp.bfloat16),
                preferred_element_type=jnp.float32)
    m = m + bc_ref[...]
    xy = m[:, :256] + rp_ref[0]
    logits = m[:, 256:]

    loc = jnp.clip(xy, 0.0, 1.0) * scale_ref[...] - 0.5
    t0 = jnp.floor(loc)
    c0 = jnp.clip(t0, 0.0, maxc_ref[...])
    c1 = jnp.clip(t0 + 1.0, 0.0, maxc_ref[...])
    wa = c1 - loc
    wb = loc - c0

    x0 = c0[:, :128]
    y0 = c0[:, 128:]
    x1 = c1[:, :128]
    y1 = c1[:, 128:]
    ax = wa[:, :128]
    ay = wa[:, 128:]
    bx = wb[:, :128]
    by = wb[:, 128:]

    mx = jnp.max(logits, axis=-1, keepdims=True)
    e = jnp.exp(logits - mx)
    gs = jnp.dot(e, g_ref[...], preferred_element_type=jnp.float32)
    aw = e / gs

    sl = sl_ref[...]
    wl = wl_ref[...]
    ra = (sl + y0 * wl + x0) * 16.0
    rb = (sl + y1 * wl + x0) * 16.0
    rc = (sl + y0 * wl + x1) * 16.0
    rd = (sl + y1 * wl + x1) * 16.0
    pa = ax * ay * aw
    pb = ax * by * aw
    pc = bx * ay * aw
    pd = bx * by * aw

    corners = [(ra, pa), (rb, pb), (rc, pc), (rd, pd)]
    for h in range(_NH):
        s = slice(h * 16, (h + 1) * 16)
        for ci, (r, p) in enumerate(corners):
            idx_ref[0, h, :, ci * 16:(ci + 1) * 16] = r[:, s].astype(jnp.int32)
            wt_ref[0, h, :, ci * 16:(ci + 1) * 16] = p[:, s]


def _t2(query, rp_bcast, wc, bc, g, scale, maxc, sl, wl):
    return pl.pallas_call(
        _t2_body,
        grid=(_B, _NRB),
        in_specs=[
            pl.BlockSpec((1, _RB, _D), lambda b, r: (b, r, 0)),
            pl.BlockSpec((1, _RB, 256), lambda b, r: (b, r, 0)),
            pl.BlockSpec((_D, 384), lambda b, r: (0, 0)),
            pl.BlockSpec((1, 384), lambda b, r: (0, 0)),
            pl.BlockSpec((128, 128), lambda b, r: (0, 0)),
            pl.BlockSpec((1, 256), lambda b, r: (0, 0)),
            pl.BlockSpec((1, 256), lambda b, r: (0, 0)),
            pl.BlockSpec((1, 128), lambda b, r: (0, 0)),
            pl.BlockSpec((1, 128), lambda b, r: (0, 0)),
        ],
        out_specs=[
            pl.BlockSpec((1, _NH, _RB, _NK), lambda b, r: (b, 0, r, 0)),
            pl.BlockSpec((1, _NH, _RB, _NK), lambda b, r: (b, 0, r, 0)),
        ],
        out_shape=[
            jax.ShapeDtypeStruct((_B, _NH, _LQ, _NK), jnp.int32),
            jax.ShapeDtypeStruct((_B, _NH, _LQ, _NK), jnp.float32),
        ],
    )(query, rp_bcast, wc, bc, g, scale, maxc, sl, wl)


def _sc_body(tbl, idx, wt, out, tblv, idxv, wtv, outv):
    c = lax.axis_index("core")
    s = lax.axis_index("sub")
    b = c
    h = s // 2
    half = s % 2

    pltpu.sync_copy(tbl.at[b, h], tblv)

    iota16 = lax.iota(jnp.int32, 16)
    sh16 = jnp.full((16,), 16, jnp.int32)
    mask_hi = jnp.full((16,), -65536, jnp.int32)
    half_ulp = jnp.full((16,), 32768, jnp.int32)
    zf32 = jnp.zeros((16,), jnp.float32)
    zi32 = jnp.zeros((16,), jnp.int32)
    rots = [(iota16 + j) & 15 for j in range(16)]

    rots8 = [(iota16 + j) & 7 for j in range(8)]

    def chunk_body(ch, carry):
        qs = half * _QHALF + ch * _CHQ
        pltpu.sync_copy(idx.at[b, h, pl.ds(qs * _NK, _CHQ * _NK)], idxv)
        pltpu.sync_copy(wt.at[b, h, pl.ds(qs * _NK, _CHQ * _NK)], wtv)

        def qb_body(qb, carry2):
            qv = iota16 + qb * 16
            qv64 = qv * _NK
            for d in range(_HD):
                plsc.store_scatter(outv, [qv, zi32 + d], zf32)

            def kg_body(kg, carry3):
                kbase = kg * 8
                addrs = [qv64 + (rots8[j] + kbase) for j in range(8)]
                idxrs = [plsc.load_gather(idxv, [a]) for a in addrs]
                wtrs = [plsc.load_gather(wtv, [a]) for a in addrs]
                wtbs = []
                for w in wtrs:
                    wbits = plsc.bitcast(w, jnp.int32)
                    r = (wbits + half_ulp) & mask_hi
                    word = r | lax.shift_right_logical(r, sh16)
                    wtbs.append(plsc.bitcast(word, jnp.bfloat16))
                for grp in range(2):
                    acc = [None] * 16
                    for jj in range(_FL):
                        j = grp * _FL + jj
                        wvs = [plsc.load_gather(tblv, [idxrs[j] + rots[s]])
                               for s in range(16)]
                        for s in range(16):
                            t = wtbs[j] * plsc.bitcast(wvs[s], jnp.bfloat16)
                            acc[s] = t if jj == 0 else acc[s] + t
                    for s in range(16):
                        u = plsc.bitcast(acc[s], jnp.int32)
                        fhi = plsc.bitcast(u & mask_hi, jnp.float32)
                        flo = plsc.bitcast(lax.shift_left(u, sh16),
                                           jnp.float32)
                        col = rots[s] + rots[s]
                        plsc.addupdate_scatter(outv, [qv, col], flo)
                        plsc.addupdate_scatter(outv, [qv, col + 1], fhi)
                return carry3

            lax.fori_loop(0, _NK // 8, kg_body, 0)
            return carry2

        lax.fori_loop(0, _CHQ // 16, qb_body, 0)
        pltpu.sync_copy(outv, out.at[b, h, pl.ds(qs, _CHQ)])
        return carry

    lax.fori_loop(0, _NCH, chunk_body, 0)


@functools.cache
def _sc_sample():
    return pl.kernel(
        _sc_body,
        out_type=jax.ShapeDtypeStruct((_B, _NH, _LQ, 33), jnp.float32),
        mesh=plsc.VectorSubcoreMesh(core_axis_name="core",
                                    subcore_axis_name="sub",
                                    num_cores=2, num_subcores=16),
        compiler_params=pltpu.CompilerParams(needs_layout_passes=False),
        scratch_types=[
            pltpu.VMEM((_LEN * 16,), jnp.int32),
            pltpu.VMEM((_CHQ * _NK,), jnp.int32),
            pltpu.VMEM((_CHQ * _NK,), jnp.float32),
            pltpu.VMEM((_CHQ, 33), jnp.float32),
        ],
    )


def _t3_body(s_ref, w_ref, b_ref, out_ref):
    h = pl.program_id(2)
    acc = jnp.dot(s_ref[0, 0, :, :_HD].astype(jnp.bfloat16),
                  w_ref[...].astype(jnp.bfloat16),
                  preferred_element_type=jnp.float32)

    @pl.when(h == 0)
    def _():
        out_ref[0] = acc + b_ref[...]

    @pl.when(h != 0)
    def _():
        out_ref[0] = out_ref[0] + acc


def _t3(sampled, wo_t, b_o):
    return pl.pallas_call(
        _t3_body,
        grid=(_B, _NRB, _NH),
        in_specs=[
            pl.BlockSpec((1, 1, _RB, 33), lambda b, r, h: (b, h, r, 0)),
            pl.BlockSpec((_HD, _D), lambda b, r, h: (h, 0)),
            pl.BlockSpec((1, _D), lambda b, r, h: (0, 0)),
        ],
        out_specs=pl.BlockSpec((1, _RB, _D), lambda b, r, h: (b, r, 0)),
        out_shape=jax.ShapeDtypeStruct((_B, _LQ, _D), jnp.float32),
    )(sampled, wo_t, b_o.reshape(1, _D))


def _consts():
    perm = np.empty((256,), np.int64)
    for half in range(2):
        for h in range(_NH):
            for dp in range(16):
                perm[half * 128 + h * 16 + dp] = h * 32 + 2 * dp + half

    lanes = np.arange(128)
    lvl = (lanes % 16) // 4
    w_l = np.array([_SPATIAL[l][1] for l in lvl], np.float32)
    h_l = np.array([_SPATIAL[l][0] for l in lvl], np.float32)
    s_l = np.array([_STARTS[l] for l in lvl], np.float32)

    scale = np.concatenate([w_l, h_l]).reshape(1, 256)
    maxc = np.concatenate([w_l - 1, h_l - 1]).reshape(1, 256)
    sl = s_l.reshape(1, 128)
    wl = w_l.reshape(1, 128)

    scm = np.zeros((8, 384), np.float32)
    for lane in range(128):
        l = int(lvl[lane])
        scm[2 * l, lane] = 1.0
        scm[2 * l + 1, 128 + lane] = 1.0

    g = (np.arange(128)[:, None] // 16 == np.arange(128)[None, :] // 16)
    g = g.astype(np.float32)
    return perm, scale, maxc, sl, wl, scm, g


_PERM, _SCALE, _MAXC, _SL, _WL, _SCM, _G = _consts()


def kernel(query, reference_points, input_flatten, input_spatial_shapes,
           input_level_start_index, W_so, b_so, W_aw, b_aw, W_v, b_v,
           W_o, b_o):
    w_perm_t = W_v[_PERM, :].T
    b_perm = b_v[_PERM].reshape(1, _D)
    wc = jnp.concatenate([W_so[0::2], W_so[1::2], W_aw], axis=0).T
    bc = jnp.concatenate([b_so[0::2], b_so[1::2], b_aw]).reshape(1, 384)
    wo_t = W_o.T
    rp4 = jnp.repeat(reference_points, _NP, axis=2)
    rp_bcast = jnp.concatenate(
        [jnp.tile(rp4[..., 0], (1, 1, _NH)),
         jnp.tile(rp4[..., 1], (1, 1, _NH))], axis=-1)

    tblp = _t1(input_flatten, w_perm_t, b_perm)
    idx_a, wt_a = _t2(query, rp_bcast, wc, bc,
                      jnp.asarray(_G), jnp.asarray(_SCALE),
                      jnp.asarray(_MAXC), jnp.asarray(_SL), jnp.asarray(_WL))
    sampled = _sc_sample()(tblp.reshape(_B, _NH, _LEN * 16),
                           idx_a.reshape(_B, _NH, _LQ * _NK),
                           wt_a.reshape(_B, _NH, _LQ * _NK))
    return _t3(sampled, wo_t, b_o)


# speedup vs baseline: 1.1284x; 1.0363x over previous
import functools

import numpy as np
import jax
import jax.numpy as jnp
from jax import lax
from jax.experimental import pallas as pl
from jax.experimental.pallas import tpu as pltpu
from jax.experimental.pallas import tpu_sc as plsc

_D = 256
_NH = 8
_NL = 4
_NP = 4
_HD = _D // _NH
_NPTS = _NL * _NP
_NK = _NPTS * 4
_SPATIAL = [(64, 64), (32, 32), (16, 16), (8, 8)]
_STARTS = [0, 4096, 5120, 5376]
_LEN = 5440
_B = 2
_LQ = 5440

_RB = 544
_NRB = _LQ // _RB

_CHQ = 160
_FL = 4
_QHALF = _LQ // 2
_NCH = _QHALF // _CHQ


def _t1_body(x_ref, w_ref, b_ref, out_ref):
    v = jnp.dot(x_ref[0].astype(jnp.bfloat16), w_ref[...].astype(jnp.bfloat16),
                preferred_element_type=jnp.float32)
    v = v + b_ref[...]
    lo = v[:, :128]
    hi = v[:, 128:]
    bl = lax.bitcast_convert_type(lo, jnp.int32)
    bh = lax.bitcast_convert_type(hi, jnp.int32)
    mask = jnp.int32(-65536)
    sh16 = jnp.full(bl.shape, 16, jnp.int32)
    rl = (bl + 32768) & mask
    rh = (bh + 32768) & mask
    word = rh | lax.shift_right_logical(rl, sh16)
    for h in range(_NH):
        out_ref[0, h, :, :] = word[:, h * 16:(h + 1) * 16]


def _t1(input_flatten, w_perm_t, b_perm):
    return pl.pallas_call(
        _t1_body,
        grid=(_B, _LEN // _RB),
        in_specs=[
            pl.BlockSpec((1, _RB, _D), lambda b, r: (b, r, 0)),
            pl.BlockSpec((_D, _D), lambda b, r: (0, 0)),
            pl.BlockSpec((1, _D), lambda b, r: (0, 0)),
        ],
        out_specs=pl.BlockSpec((1, _NH, _RB, 16), lambda b, r: (b, 0, r, 0)),
        out_shape=jax.ShapeDtypeStruct((_B, _NH, _LEN, 16), jnp.int32),
    )(input_flatten, w_perm_t, b_perm)


def _t2_body(q_ref, rp_ref, wc_ref, bc_ref, sc_ref, g_ref,
             scale_ref, maxc_ref, sl_ref, wl_ref, idx_ref, wt_ref):
    m = jnp.dot(q_ref[0].astype(jnp.bfloat16), wc_ref[...].astype(jnp.bfloat16),
                preferred_element_type=jnp.float32)
    m = m + bc_ref[...]
    rpb = jax.lax.dot_general(rp_ref[0], sc_ref[...],
                              (((1,), (0,)), ((), ())),
                              precision=jax.lax.Precision.HIGHEST,
                              preferred_element_type=jnp.float32)
    xy = m[:, :256] + rpb
    logits = m[:, 256:]

    loc = jnp.clip(xy, 0.0, 1.0) * scale_ref[...] - 0.5
    t0 = jnp.floor(loc)
    c0 = jnp.clip(t0, 0.0, maxc_ref[...])
    c1 = jnp.clip(t0 + 1.0, 0.0, maxc_ref[...])
    wa = c1 - loc
    wb = loc - c0

    x0 = c0[:, :128]
    y0 = c0[:, 128:]
    x1 = c1[:, :128]
    y1 = c1[:, 128:]
    ax = wa[:, :128]
    ay = wa[:, 128:]
    bx = wb[:, :128]
    by = wb[:, 128:]

    mx = jnp.max(logits, axis=-1, keepdims=True)
    e = jnp.exp(logits - mx)
    gs = jnp.dot(e, g_ref[...], preferred_element_type=jnp.float32)
    aw = e / gs

    sl = sl_ref[...]
    wl = wl_ref[...]
    ra = (sl + y0 * wl + x0) * 16.0
    rb = (sl + y1 * wl + x0) * 16.0
    rc = (sl + y0 * wl + x1) * 16.0
    rd = (sl + y1 * wl + x1) * 16.0
    pa = ax * ay * aw
    pb = ax * by * aw
    pc = bx * ay * aw
    pd = bx * by * aw

    corners = [(ra, pa), (rb, pb), (rc, pc), (rd, pd)]
    for h in range(_NH):
        s = slice(h * 16, (h + 1) * 16)
        for ci, (r, p) in enumerate(corners):
            idx_ref[0, h, :, ci * 16:(ci + 1) * 16] = r[:, s].astype(jnp.int32)
            wt_ref[0, h, :, ci * 16:(ci + 1) * 16] = p[:, s]


def _t2(query, rp_flat, wc, bc, sc, g, scale, maxc, sl, wl):
    return pl.pallas_call(
        _t2_body,
        grid=(_B, _NRB),
        in_specs=[
            pl.BlockSpec((1, _RB, _D), lambda b, r: (b, r, 0)),
            pl.BlockSpec((1, _RB, 8), lambda b, r: (b, r, 0)),
            pl.BlockSpec((_D, 384), lambda b, r: (0, 0)),
            pl.BlockSpec((1, 384), lambda b, r: (0, 0)),
            pl.BlockSpec((8, 256), lambda b, r: (0, 0)),
            pl.BlockSpec((128, 128), lambda b, r: (0, 0)),
            pl.BlockSpec((1, 256), lambda b, r: (0, 0)),
            pl.BlockSpec((1, 256), lambda b, r: (0, 0)),
            pl.BlockSpec((1, 128), lambda b, r: (0, 0)),
            pl.BlockSpec((1, 128), lambda b, r: (0, 0)),
        ],
        out_specs=[
            pl.BlockSpec((1, _NH, _RB, _NK), lambda b, r: (b, 0, r, 0)),
            pl.BlockSpec((1, _NH, _RB, _NK), lambda b, r: (b, 0, r, 0)),
        ],
        out_shape=[
            jax.ShapeDtypeStruct((_B, _NH, _LQ, _NK), jnp.int32),
            jax.ShapeDtypeStruct((_B, _NH, _LQ, _NK), jnp.float32),
        ],
    )(query, rp_flat, wc, bc, sc, g, scale, maxc, sl, wl)


def _sc_body(tbl, idx, wt, out, tblv, idxv, wtv, outv):
    c = lax.axis_index("core")
    s = lax.axis_index("sub")
    b = c
    h = s // 2
    half = s % 2

    pltpu.sync_copy(tbl.at[b, h], tblv)

    iota16 = lax.iota(jnp.int32, 16)
    sh16 = jnp.full((16,), 16, jnp.int32)
    mask_hi = jnp.full((16,), -65536, jnp.int32)
    half_ulp = jnp.full((16,), 32768, jnp.int32)
    zf32 = jnp.zeros((16,), jnp.float32)
    zi32 = jnp.zeros((16,), jnp.int32)
    rots = [(iota16 + j) & 15 for j in range(16)]

    rots8 = [(iota16 + j) & 7 for j in range(8)]

    def chunk_body(ch, carry):
        qs = half * _QHALF + ch * _CHQ
        pltpu.sync_copy(idx.at[b, h, pl.ds(qs * _NK, _CHQ * _NK)], idxv)
        pltpu.sync_copy(wt.at[b, h, pl.ds(qs * _NK, _CHQ * _NK)], wtv)

        def qb_body(qb, carry2):
            qv = iota16 + qb * 16
            qv64 = qv * _NK
            for d in range(_HD):
                plsc.store_scatter(outv, [qv, zi32 + d], zf32)

            def kg_body(kg, carry3):
                kbase = kg * 8
                addrs = [qv64 + (rots8[j] + kbase) for j in range(8)]
                idxrs = [plsc.load_gather(idxv, [a]) for a in addrs]
                wtrs = [plsc.load_gather(wtv, [a]) for a in addrs]
                wtbs = []
                for w in wtrs:
                    wbits = plsc.bitcast(w, jnp.int32)
                    r = (wbits + half_ulp) & mask_hi
                    word = r | lax.shift_right_logical(r, sh16)
                    wtbs.append(plsc.bitcast(word, jnp.bfloat16))
                for grp in range(2):
                    acc = [None] * 16
                    for jj in range(_FL):
                        j = grp * _FL + jj
                        wvs = [plsc.load_gather(tblv, [idxrs[j] + rots[s]])
                               for s in range(16)]
                        for s in range(16):
                            t = wtbs[j] * plsc.bitcast(wvs[s], jnp.bfloat16)
                            acc[s] = t if jj == 0 else acc[s] + t
                    for s in range(16):
                        u = plsc.bitcast(acc[s], jnp.int32)
                        fhi = plsc.bitcast(u & mask_hi, jnp.float32)
                        flo = plsc.bitcast(lax.shift_left(u, sh16),
                                           jnp.float32)
                        col = rots[s] + rots[s]
                        plsc.addupdate_scatter(outv, [qv, col], flo)
                        plsc.addupdate_scatter(outv, [qv, col + 1], fhi)
                return carry3

            lax.fori_loop(0, _NK // 8, kg_body, 0)
            return carry2

        lax.fori_loop(0, _CHQ // 16, qb_body, 0)
        pltpu.sync_copy(outv, out.at[b, h, pl.ds(qs, _CHQ)])
        return carry

    lax.fori_loop(0, _NCH, chunk_body, 0)


@functools.cache
def _sc_sample():
    return pl.kernel(
        _sc_body,
        out_type=jax.ShapeDtypeStruct((_B, _NH, _LQ, 33), jnp.float32),
        mesh=plsc.VectorSubcoreMesh(core_axis_name="core",
                                    subcore_axis_name="sub",
                                    num_cores=2, num_subcores=16),
        compiler_params=pltpu.CompilerParams(needs_layout_passes=False),
        scratch_types=[
            pltpu.VMEM((_LEN * 16,), jnp.int32),
            pltpu.VMEM((_CHQ * _NK,), jnp.int32),
            pltpu.VMEM((_CHQ * _NK,), jnp.float32),
            pltpu.VMEM((_CHQ, 33), jnp.float32),
        ],
    )


def _t3_body(s_ref, w_ref, b_ref, out_ref):
    h = pl.program_id(2)
    acc = jnp.dot(s_ref[0, 0, :, :_HD].astype(jnp.bfloat16),
                  w_ref[...].astype(jnp.bfloat16),
                  preferred_element_type=jnp.float32)

    @pl.when(h == 0)
    def _():
        out_ref[0] = acc + b_ref[...]

    @pl.when(h != 0)
    def _():
        out_ref[0] = out_ref[0] + acc


def _t3(sampled, wo_t, b_o):
    return pl.pallas_call(
        _t3_body,
        grid=(_B, _NRB, _NH),
        in_specs=[
            pl.BlockSpec((1, 1, _RB, 33), lambda b, r, h: (b, h, r, 0)),
            pl.BlockSpec((_HD, _D), lambda b, r, h: (h, 0)),
            pl.BlockSpec((1, _D), lambda b, r, h: (0, 0)),
        ],
        out_specs=pl.BlockSpec((1, _RB, _D), lambda b, r, h: (b, r, 0)),
        out_shape=jax.ShapeDtypeStruct((_B, _LQ, _D), jnp.float32),
    )(sampled, wo_t, b_o.reshape(1, _D))


def _consts():
    perm = np.empty((256,), np.int64)
    for half in range(2):
        for h in range(_NH):
            for dp in range(16):
                perm[half * 128 + h * 16 + dp] = h * 32 + 2 * dp + half

    lanes = np.arange(128)
    lvl = (lanes % 16) // 4
    w_l = np.array([_SPATIAL[l][1] for l in lvl], np.float32)
    h_l = np.array([_SPATIAL[l][0] for l in lvl], np.float32)
    s_l = np.array([_STARTS[l] for l in lvl], np.float32)

    scale = np.concatenate([w_l, h_l]).reshape(1, 256)
    maxc = np.concatenate([w_l - 1, h_l - 1]).reshape(1, 256)
    sl = s_l.reshape(1, 128)
    wl = w_l.reshape(1, 128)

    scm = np.zeros((8, 256), np.float32)
    for lane in range(128):
        l = int(lvl[lane])
        scm[2 * l, lane] = 1.0
        scm[2 * l + 1, 128 + lane] = 1.0

    g = (np.arange(128)[:, None] // 16 == np.arange(128)[None, :] // 16)
    g = g.astype(np.float32)
    return perm, scale, maxc, sl, wl, scm, g


_PERM, _SCALE, _MAXC, _SL, _WL, _SCM, _G = _consts()


def kernel(query, reference_points, input_flatten, input_spatial_shapes,
           input_level_start_index, W_so, b_so, W_aw, b_aw, W_v, b_v,
           W_o, b_o):
    w_perm_t = W_v[_PERM, :].T
    b_perm = b_v[_PERM].reshape(1, _D)
    wc = jnp.concatenate([W_so[0::2], W_so[1::2], W_aw], axis=0).T
    bc = jnp.concatenate([b_so[0::2], b_so[1::2], b_aw]).reshape(1, 384)
    wo_t = W_o.T
    rp_flat = reference_points.reshape(_B, _LQ, _NL * 2)

    tblp = _t1(input_flatten, w_perm_t, b_perm)
    idx_a, wt_a = _t2(query, rp_flat, wc, bc, jnp.asarray(_SCM),
                      jnp.asarray(_G), jnp.asarray(_SCALE),
                      jnp.asarray(_MAXC), jnp.asarray(_SL), jnp.asarray(_WL))
    sampled = _sc_sample()(tblp.reshape(_B, _NH, _LEN * 16),
                           idx_a.reshape(_B, _NH, _LQ * _NK),
                           wt_a.reshape(_B, _NH, _LQ * _NK))
    return _t3(sampled, wo_t, b_o)
